# baseline (speedup 1.0000x reference)
import jax
import jax.numpy as jnp
from jax.experimental import pallas as pl
from jax.experimental.pallas import tpu as pltpu

H = 2048
E = 64
TOP_K = 2
N_TOK = 8192
BT = 1024
NBLK = N_TOK // BT


def _main_kernel(x_ref, wg_ref, w1_ref, b1_ref, w2_ref, b2_ref, ebias_ref,
                 ts_ref, ti_ref, loss_ref, cnt_acc, ps_acc, cs_acc):
    i = pl.program_id(0)
    x = x_ref[...]
    dims = (((1,), (1,)), ((), ()))
    logits = jax.lax.dot_general(wg_ref[...], x, dims,
                                 preferred_element_type=jnp.float32)
    h1pre = jax.lax.dot_general(w1_ref[...], x, dims,
                                preferred_element_type=jnp.float32)

    m = jnp.max(logits, axis=0, keepdims=True)
    ex = jnp.exp(logits - m)
    scores = ex / jnp.sum(ex, axis=0, keepdims=True)

    biased = scores + ebias_ref[...]
    iota = jax.lax.broadcasted_iota(jnp.int32, (E, BT), 0)
    m1 = jnp.max(biased, axis=0, keepdims=True)
    sel1 = iota == jnp.min(jnp.where(biased == m1, iota, E),
                           axis=0, keepdims=True)
    masked = jnp.where(sel1, -jnp.inf, biased)
    m2 = jnp.max(masked, axis=0, keepdims=True)
    sel2 = iota == jnp.min(jnp.where(masked == m2, iota, E),
                           axis=0, keepdims=True)

    s1 = jnp.sum(jnp.where(sel1, scores, 0.0), axis=0, keepdims=True)
    s2 = jnp.sum(jnp.where(sel2, scores, 0.0), axis=0, keepdims=True)
    rden = 1.0 / (s1 + s2)
    ts_ref[...] = jnp.concatenate([s1 * rden, s2 * rden], axis=0)
    ti_ref[...] = jnp.concatenate(
        [jnp.sum(jnp.where(sel1, iota, 0), axis=0, keepdims=True),
         jnp.sum(jnp.where(sel2, iota, 0), axis=0, keepdims=True)], axis=0)

    h1 = jnp.maximum(h1pre + b1_ref[...], 0.0)
    c = jax.nn.sigmoid(jnp.sum(h1 * w2_ref[...], axis=0, keepdims=True)
                       + b2_ref[...])

    cnt_blk = jnp.sum(sel1.astype(jnp.float32) + sel2.astype(jnp.float32),
                      axis=1, keepdims=True)
    ps_blk = jnp.sum(scores, axis=1, keepdims=True)
    cs_blk = jnp.sum(c).reshape(1, 1)

    keep = i > 0
    cnt = jnp.where(keep, cnt_acc[...], 0.0) + cnt_blk
    ps = jnp.where(keep, ps_acc[...], 0.0) + ps_blk
    cs = jnp.where(keep, cs_acc[...], 0.0) + cs_blk
    cnt_acc[...] = cnt
    ps_acc[...] = ps
    cs_acc[...] = cs
    aux = E * jnp.sum(cnt * ps) / (N_TOK * TOP_K * N_TOK)
    loss_ref[...] = aux * (0.5 + cs / N_TOK)


@jax.jit
def kernel(hidden_states, Wg, W1, b1, W2, b2, expert_bias):
    x = hidden_states.reshape(-1, H)
    b1r = b1.reshape(E, 1)
    w2r = W2.reshape(E, 1)
    b2r = b2.reshape(1, 1)
    ebr = expert_bias.reshape(E, 1)

    ts, ti, loss = pl.pallas_call(
        _main_kernel,
        grid=(NBLK,),
        in_specs=[
            pl.BlockSpec((BT, H), lambda i: (i, 0)),
            pl.BlockSpec((E, H), lambda i: (0, 0)),
            pl.BlockSpec((E, H), lambda i: (0, 0)),
            pl.BlockSpec((E, 1), lambda i: (0, 0)),
            pl.BlockSpec((E, 1), lambda i: (0, 0)),
            pl.BlockSpec((1, 1), lambda i: (0, 0)),
            pl.BlockSpec((E, 1), lambda i: (0, 0)),
        ],
        out_specs=[
            pl.BlockSpec((TOP_K, BT), lambda i: (0, i)),
            pl.BlockSpec((TOP_K, BT), lambda i: (0, i)),
            pl.BlockSpec((1, 1), lambda i: (0, 0)),
        ],
        out_shape=[
            jax.ShapeDtypeStruct((TOP_K, N_TOK), jnp.float32),
            jax.ShapeDtypeStruct((TOP_K, N_TOK), jnp.int32),
            jax.ShapeDtypeStruct((1, 1), jnp.float32),
        ],
        scratch_shapes=[
            pltpu.VMEM((E, 1), jnp.float32),
            pltpu.VMEM((E, 1), jnp.float32),
            pltpu.VMEM((1, 1), jnp.float32),
        ],
        compiler_params=pltpu.CompilerParams(
            dimension_semantics=("arbitrary",)),
    )(x, Wg, W1, b1r, w2r, b2r, ebr)

    return ts.T, ti.T, loss.reshape(())


# speedup vs baseline: 1.0376x; 1.0376x over previous
import jax
import jax.numpy as jnp
from jax.experimental import pallas as pl
from jax.experimental.pallas import tpu as pltpu

H = 2048
E = 64
TOP_K = 2
N_TOK = 8192
BT = 1024
NBLK = N_TOK // BT


def _main_kernel(x_ref, wc_ref, b1_ref, w2_ref, b2_ref, ebias_ref,
                 ts_ref, ti_ref, loss_ref, cnt_acc, ps_acc, cs_acc):
    i = pl.program_id(0)
    x = x_ref[...]
    both_t = jax.lax.dot_general(
        wc_ref[...], x, (((1,), (1,)), ((), ())),
        preferred_element_type=jnp.float32)
    logits = both_t[:E]
    h1pre = both_t[E:]

    m = jnp.max(logits, axis=0, keepdims=True)
    ex = jnp.exp(logits - m)
    scores = ex / jnp.sum(ex, axis=0, keepdims=True)

    biased = scores + ebias_ref[...]
    iota = jax.lax.broadcasted_iota(jnp.int32, (E, BT), 0)
    m1 = jnp.max(biased, axis=0, keepdims=True)
    sel1 = iota == jnp.min(jnp.where(biased == m1, iota, E),
                           axis=0, keepdims=True)
    masked = jnp.where(sel1, -jnp.inf, biased)
    m2 = jnp.max(masked, axis=0, keepdims=True)
    sel2 = iota == jnp.min(jnp.where(masked == m2, iota, E),
                           axis=0, keepdims=True)

    s1 = jnp.sum(jnp.where(sel1, scores, 0.0), axis=0, keepdims=True)
    s2 = jnp.sum(jnp.where(sel2, scores, 0.0), axis=0, keepdims=True)
    rden = 1.0 / (s1 + s2)
    ts_ref[...] = jnp.concatenate([s1 * rden, s2 * rden], axis=0)
    ti_ref[...] = jnp.concatenate(
        [jnp.sum(jnp.where(sel1, iota, 0), axis=0, keepdims=True),
         jnp.sum(jnp.where(sel2, iota, 0), axis=0, keepdims=True)], axis=0)

    h1 = jnp.maximum(h1pre + b1_ref[...], 0.0)
    c = jax.nn.sigmoid(jnp.sum(h1 * w2_ref[...], axis=0, keepdims=True)
                       + b2_ref[...])

    cnt_blk = jnp.sum(sel1.astype(jnp.float32) + sel2.astype(jnp.float32),
                      axis=1, keepdims=True)
    ps_blk = jnp.sum(scores, axis=1, keepdims=True)
    cs_blk = jnp.sum(c).reshape(1, 1)

    keep = i > 0
    cnt = jnp.where(keep, cnt_acc[...], 0.0) + cnt_blk
    ps = jnp.where(keep, ps_acc[...], 0.0) + ps_blk
    cs = jnp.where(keep, cs_acc[...], 0.0) + cs_blk
    cnt_acc[...] = cnt
    ps_acc[...] = ps
    cs_acc[...] = cs
    aux = E * jnp.sum(cnt * ps) / (N_TOK * TOP_K * N_TOK)
    loss_ref[...] = aux * (0.5 + cs / N_TOK)


@jax.jit
def kernel(hidden_states, Wg, W1, b1, W2, b2, expert_bias):
    x = hidden_states.reshape(-1, H)
    wc = jnp.concatenate([Wg, W1], axis=0)
    b1r = b1.reshape(E, 1)
    w2r = W2.reshape(E, 1)
    b2r = b2.reshape(1, 1)
    ebr = expert_bias.reshape(E, 1)

    ts, ti, loss = pl.pallas_call(
        _main_kernel,
        grid=(NBLK,),
        in_specs=[
            pl.BlockSpec((BT, H), lambda i: (i, 0)),
            pl.BlockSpec((2 * E, H), lambda i: (0, 0)),
            pl.BlockSpec((E, 1), lambda i: (0, 0)),
            pl.BlockSpec((E, 1), lambda i: (0, 0)),
            pl.BlockSpec((1, 1), lambda i: (0, 0)),
            pl.BlockSpec((E, 1), lambda i: (0, 0)),
        ],
        out_specs=[
            pl.BlockSpec((TOP_K, BT), lambda i: (0, i)),
            pl.BlockSpec((TOP_K, BT), lambda i: (0, i)),
            pl.BlockSpec((1, 1), lambda i: (0, 0)),
        ],
        out_shape=[
            jax.ShapeDtypeStruct((TOP_K, N_TOK), jnp.float32),
            jax.ShapeDtypeStruct((TOP_K, N_TOK), jnp.int32),
            jax.ShapeDtypeStruct((1, 1), jnp.float32),
        ],
        scratch_shapes=[
            pltpu.VMEM((E, 1), jnp.float32),
            pltpu.VMEM((E, 1), jnp.float32),
            pltpu.VMEM((1, 1), jnp.float32),
        ],
        compiler_params=pltpu.CompilerParams(
            dimension_semantics=("arbitrary",)),
    )(x, wc, b1r, w2r, b2r, ebr)

    return ts.T, ti.T, loss.reshape(())
